# baseline (speedup 1.0000x reference)
import functools

import jax
import jax.numpy as jnp
from jax import lax
from jax.experimental import pallas as pl
from jax.experimental.pallas import tpu as pltpu
from jax.experimental.pallas import tpu_sc as plsc

NC = 2
NS = 16
NW = NC * NS

f32 = jnp.float32
i32 = jnp.int32


@functools.lru_cache(maxsize=None)
def _make_sc_degrees(n_pad, n_edges, block):
    e_per_w = n_edges // NW
    n_full = e_per_w // block
    tail = e_per_w - n_full * block
    assert block % 16 == 0 and block <= 128 and n_full % 2 == 0
    assert tail % 8 == 0 and (tail == 0 or tail >= 8)
    rpt = n_pad // NS
    assert rpt % 8 == 0
    mesh = plsc.VectorSubcoreMesh(
        core_axis_name="c", subcore_axis_name="s", num_cores=NC, num_subcores=NS
    )

    @functools.partial(
        pl.kernel,
        out_type=jax.ShapeDtypeStruct((NC * 2 * n_pad,), f32),
        mesh=mesh,
        scratch_types=[
            pltpu.VMEM_SHARED((n_pad,), f32),
            pltpu.VMEM_SHARED((n_pad,), f32),
            pltpu.VMEM((block,), f32),
            pltpu.VMEM((block,), i32),
            pltpu.VMEM((block,), i32),
            pltpu.VMEM((block,), i32),
            pltpu.VMEM((block,), i32),
            pltpu.VMEM((max(tail, 8),), i32),
            pltpu.VMEM((max(tail, 8),), i32),
            pltpu.VMEM((rpt,), f32),
            pltpu.SemaphoreType.DMA,
        ],
    )
    def deg_kernel(src_hbm, dst_hbm, zeros_hbm, out_hbm,
                   acc_out, acc_in, ones, sidx0, didx0, sidx1, didx1,
                   sidx_t, didx_t, zbuf, sem_i):
        c = lax.axis_index("c")
        s = lax.axis_index("s")
        wid = s * NC + c

        for j in range(block // 16):
            ones[pl.ds(j * 16, 16)] = jnp.full((16,), 1.0, f32)

        pltpu.sync_copy(zeros_hbm, zbuf)
        pltpu.sync_copy(zbuf, acc_out.at[pl.ds(s * rpt, rpt)])
        pltpu.sync_copy(zbuf, acc_in.at[pl.ds(s * rpt, rpt)])
        plsc.subcore_barrier()

        ebase = wid * e_per_w

        def idx_load(i, sbuf, dbuf):
            base = pl.multiple_of(ebase + i * block, 8)
            a = pltpu.async_copy(src_hbm.at[pl.ds(base, block)], sbuf, sem_i)
            b = pltpu.async_copy(dst_hbm.at[pl.ds(base, block)], dbuf, sem_i)
            return a, b

        a, b = idx_load(0, sidx0, didx0)
        a.wait()
        b.wait()

        @pl.loop(0, n_full // 2)
        def _(p):
            i0 = 2 * p
            a, b = idx_load(i0 + 1, sidx1, didx1)
            pltpu.sync_copy(ones, acc_out.at[sidx0], add=True)
            pltpu.sync_copy(ones, acc_in.at[didx0], add=True)
            a.wait()
            b.wait()

            @pl.when(p < n_full // 2 - 1)
            def _():
                a2, b2 = idx_load(i0 + 2, sidx0, didx0)
                pltpu.sync_copy(ones, acc_out.at[sidx1], add=True)
                pltpu.sync_copy(ones, acc_in.at[didx1], add=True)
                a2.wait()
                b2.wait()

            @pl.when(p == n_full // 2 - 1)
            def _():
                pltpu.sync_copy(ones, acc_out.at[sidx1], add=True)
                pltpu.sync_copy(ones, acc_in.at[didx1], add=True)

        if tail:
            base = pl.multiple_of(ebase + n_full * block, 8)
            pltpu.sync_copy(src_hbm.at[pl.ds(base, tail)], sidx_t)
            pltpu.sync_copy(dst_hbm.at[pl.ds(base, tail)], didx_t)
            pltpu.sync_copy(ones.at[pl.ds(0, tail)], acc_out.at[sidx_t], add=True)
            pltpu.sync_copy(ones.at[pl.ds(0, tail)], acc_in.at[didx_t], add=True)

        plsc.subcore_barrier()
        obase = pl.multiple_of((c * 2) * n_pad + s * rpt, 8)
        pltpu.sync_copy(acc_out.at[pl.ds(s * rpt, rpt)], zbuf)
        pltpu.sync_copy(zbuf, out_hbm.at[pl.ds(obase, rpt)])
        obase2 = pl.multiple_of((c * 2 + 1) * n_pad + s * rpt, 8)
        pltpu.sync_copy(acc_in.at[pl.ds(s * rpt, rpt)], zbuf)
        pltpu.sync_copy(zbuf, out_hbm.at[pl.ds(obase2, rpt)])

    return deg_kernel


@functools.lru_cache(maxsize=None)
def _make_sc_aggregate(n_pad, width, n_edges, block):
    e_per_w = n_edges // NW
    n_full = e_per_w // block
    tail = e_per_w - n_full * block
    assert block % 8 == 0 and block <= 128 and n_full % 2 == 0
    assert tail % 8 == 0 and (tail == 0 or tail >= 8)
    rpt = n_pad // NS
    nz = rpt // block
    assert nz * block == rpt
    mesh = plsc.VectorSubcoreMesh(
        core_axis_name="c", subcore_axis_name="s", num_cores=NC, num_subcores=NS
    )

    assert n_full >= 8 and (n_full - 2) % 4 == 0
    n_quads = (n_full - 2) // 4 - 1

    @functools.partial(
        pl.kernel,
        out_type=jax.ShapeDtypeStruct((NC, n_pad, width), f32),
        mesh=mesh,
        compiler_params=pltpu.CompilerParams(use_tc_tiling_on_sc=(width % 128 == 0)),
        scratch_types=[
            pltpu.VMEM_SHARED((n_pad, width), f32),
            pltpu.VMEM((block, width), f32),
            pltpu.VMEM((block, width), f32),
            pltpu.VMEM((max(tail, 8), width), f32),
            pltpu.VMEM((block,), i32),
            pltpu.VMEM((block,), i32),
            pltpu.VMEM((block,), i32),
            pltpu.VMEM((block,), i32),
            pltpu.VMEM((block,), i32),
            pltpu.VMEM((block,), i32),
            pltpu.VMEM((max(tail, 8),), i32),
            pltpu.VMEM((max(tail, 8),), i32),
            pltpu.SemaphoreType.DMA,
            pltpu.SemaphoreType.DMA,
            pltpu.SemaphoreType.DMA,
            pltpu.SemaphoreType.DMA,
            pltpu.SemaphoreType.DMA,
        ],
    )
    def agg_kernel(xs_hbm, src_hbm, dst_hbm, zeros_hbm, out_hbm,
                   acc, rows0, rows1, rows_t, sidx0, sidx1,
                   didx0, didx1, didx2, didx3,
                   sidx_t, didx_t, sem_g, sem_i0, sem_i1, sem_s0, sem_s1):
        c = lax.axis_index("c")
        s = lax.axis_index("s")
        wid = s * NC + c
        rows = [rows0, rows1]
        sidx = [sidx0, sidx1]
        didx = [didx0, didx1, didx2, didx3]
        sem_i = [sem_i0, sem_i1]
        sem_s = [sem_s0, sem_s1]

        pltpu.sync_copy(zeros_hbm, rows0)
        for j in range(nz):
            pltpu.sync_copy(rows0, acc.at[pl.ds(s * rpt + j * block, block)])
        plsc.subcore_barrier()

        ebase = wid * e_per_w

        def ld(i, k):
            base = pl.multiple_of(ebase + i * block, 8)
            pltpu.async_copy(src_hbm.at[pl.ds(base, block)], sidx[k % 2], sem_i[k % 2])
            pltpu.async_copy(dst_hbm.at[pl.ds(base, block)], didx[k % 4], sem_i[k % 2])

        def wld(k):
            pltpu.make_async_copy(src_hbm.at[pl.ds(0, block)], sidx[k % 2], sem_i[k % 2]).wait()
            pltpu.make_async_copy(dst_hbm.at[pl.ds(0, block)], didx[k % 4], sem_i[k % 2]).wait()

        def g(k):
            pltpu.async_copy(xs_hbm.at[sidx[k % 2]], rows[k % 2], sem_g)

        def wg(k):
            pltpu.make_async_copy(xs_hbm.at[sidx[k % 2]], rows[k % 2], sem_g).wait()

        def sc(k):
            pltpu.async_copy(rows[k % 2], acc.at[didx[k % 4]], sem_s[k % 2], add=True)

        def wsc(k):
            pltpu.make_async_copy(rows[k % 2], acc.at[didx[k % 4]], sem_s[k % 2]).wait()

        def body(i, k, first, last, do_ld):
            wg(k)
            sc(k)
            if not last:
                wld(k + 1)
                if not first:
                    wsc(k - 1)
                g(k + 1)
            if do_ld:
                ld(i + 2, k + 2)

        ld(0, 0)
        ld(1, 1)
        wld(0)
        g(0)
        body(0, 0, True, False, True)
        body(1, 1, False, False, True)
        body(2, 2, False, False, True)
        body(3, 3, False, False, True)

        @pl.loop(0, n_quads)
        def _(q):
            i0 = 4 * q + 4
            body(i0, 0, False, False, True)
            body(i0 + 1, 1, False, False, True)
            body(i0 + 2, 2, False, False, True)
            body(i0 + 3, 3, False, False, True)

        body(n_full - 2, 0, False, False, False)
        body(n_full - 1, 1, False, True, False)
        wsc(0)

        if tail:
            base = pl.multiple_of(ebase + n_full * block, 8)
            ta = pltpu.async_copy(src_hbm.at[pl.ds(base, tail)], sidx_t, sem_i[0])
            tb = pltpu.async_copy(dst_hbm.at[pl.ds(base, tail)], didx_t, sem_i[0])
            ta.wait()
            tb.wait()
            gt = pltpu.async_copy(xs_hbm.at[sidx_t], rows_t, sem_g)
            gt.wait()
            wsc(1)
            pltpu.sync_copy(rows_t, acc.at[didx_t], add=True)
        else:
            wsc(1)

        plsc.subcore_barrier()
        for j in range(nz):
            pltpu.sync_copy(acc.at[pl.ds(s * rpt + j * block, block)], rows0)
            pltpu.sync_copy(rows0, out_hbm.at[c, pl.ds(s * rpt + j * block, block)])

    return agg_kernel


def _tc_prep(x, d_out0, d_out1, d_in0, d_in1):
    n, d = x.shape

    def body(x_ref, a_ref, b_ref, c_ref, e_ref, xs_ref, nin_ref, nout_ref):
        deg_out = a_ref[...] + b_ref[...]
        deg_in = c_ref[...] + e_ref[...]
        nout = lax.rsqrt(jnp.maximum(deg_out, 1.0))
        nin = lax.rsqrt(jnp.maximum(deg_in, 1.0))
        xs_ref[...] = x_ref[...] * nout
        nin_ref[...] = nin
        nout_ref[...] = nout

    return pl.pallas_call(
        body,
        out_shape=(
            jax.ShapeDtypeStruct((n, d), f32),
            jax.ShapeDtypeStruct((n, 1), f32),
            jax.ShapeDtypeStruct((n, 1), f32),
        ),
    )(x, d_out0, d_out1, d_in0, d_in1)


def _tc_mid(ap, nin, nout, w1, b1, wop):
    n = nin.shape[0]
    wpad = wop.shape[1]

    def body(ap_ref, nin_ref, nout_ref, w1_ref, b1_ref, wop_ref, y_ref):
        agg = (ap_ref[0, :n, :] + ap_ref[1, :n, :]) * nin_ref[...]
        h = jnp.dot(agg, w1_ref[...], preferred_element_type=f32) + b1_ref[...]
        h = jnp.maximum(h, 0.0)
        mean = jnp.mean(h, axis=0, keepdims=True)
        var = jnp.mean(h * h, axis=0, keepdims=True) - mean * mean
        hbn = (h - mean) * lax.rsqrt(var + 1e-5)
        y2 = jnp.dot(hbn, wop_ref[...], preferred_element_type=f32)
        y_ref[...] = y2 * nout_ref[...]

    return pl.pallas_call(
        body,
        out_shape=jax.ShapeDtypeStruct((n, wpad), f32),
    )(ap, nin, nout, w1, b1, wop)


def _tc_final(a2p, nin, bfull):
    n = nin.shape[0]
    wpad = bfull.shape[1]

    def body(ap_ref, nin_ref, b_ref, out_ref):
        logits = (ap_ref[0, :n, :] + ap_ref[1, :n, :]) * nin_ref[...] + b_ref[...]
        m = jnp.max(logits, axis=1, keepdims=True)
        e = jnp.exp(logits - m)
        out_ref[...] = e / jnp.sum(e, axis=1, keepdims=True)

    return pl.pallas_call(
        body,
        out_shape=jax.ShapeDtypeStruct((n, wpad), f32),
    )(a2p, nin, bfull)


def kernel(in_feat, edge_index, W1, b1, W_out, b_out):
    n, d = in_feat.shape
    h = W1.shape[1]
    out_dim = W_out.shape[1]
    e = edge_index.shape[1]
    wpad = 16
    block = 128
    n_pad = ((n + (block * NS) - 1) // (block * NS)) * (block * NS)
    rpt = n_pad // NS

    src = edge_index[0].astype(i32)
    dst = edge_index[1].astype(i32)

    zeros1 = jnp.zeros((rpt,), f32)
    zeros_d = jnp.zeros((block, d), f32)
    zeros_w = jnp.zeros((block, wpad), f32)

    degp = _make_sc_degrees(n_pad, e, block)(src, dst, zeros1)
    degp = degp.reshape(NC, 2, n_pad)
    d_out0 = degp[0, 0, :n].reshape(n, 1)
    d_out1 = degp[1, 0, :n].reshape(n, 1)
    d_in0 = degp[0, 1, :n].reshape(n, 1)
    d_in1 = degp[1, 1, :n].reshape(n, 1)

    xs, nin, nout = _tc_prep(in_feat, d_out0, d_out1, d_in0, d_in1)

    ap = _make_sc_aggregate(n_pad, d, e, block)(xs, src, dst, zeros_d)

    wop = jnp.concatenate([W_out, jnp.zeros((h, wpad - out_dim), f32)], axis=1)
    y2s = _tc_mid(ap, nin, nout, W1, b1.reshape(1, h), wop)

    a2p = _make_sc_aggregate(n_pad, wpad, e, block)(y2s, src, dst, zeros_w)

    bfull = jnp.concatenate(
        [b_out, jnp.full((wpad - out_dim,), -1e30, f32)]
    ).reshape(1, wpad)
    out16 = _tc_final(a2p, nin, bfull)
    return out16[:, :out_dim]


# speedup vs baseline: 1.0672x; 1.0672x over previous
import functools

import jax
import jax.numpy as jnp
from jax import lax
from jax.experimental import pallas as pl
from jax.experimental.pallas import tpu as pltpu
from jax.experimental.pallas import tpu_sc as plsc

NC = 2
NS = 16
NW = NC * NS

f32 = jnp.float32
i32 = jnp.int32


@functools.lru_cache(maxsize=None)
def _make_sc_degrees(n_pad, n_edges):
    e_per_w = n_edges // NW
    assert e_per_w % 16 == 0
    rpt = n_pad // NS
    assert rpt % 16 == 0 and n_pad % 16 == 0
    mesh = plsc.VectorSubcoreMesh(
        core_axis_name="c", subcore_axis_name="s", num_cores=NC, num_subcores=NS
    )

    @functools.partial(
        pl.kernel,
        out_type=jax.ShapeDtypeStruct((NC * 2 * n_pad,), f32),
        mesh=mesh,
        compiler_params=pltpu.CompilerParams(needs_layout_passes=False),
        scratch_types=[
            pltpu.VMEM_SHARED((2, NS, n_pad), f32),
            pltpu.VMEM((n_pad,), f32),
            pltpu.VMEM((n_pad,), f32),
            pltpu.VMEM((e_per_w,), i32),
            pltpu.VMEM((e_per_w,), i32),
            pltpu.VMEM((NS, rpt), f32),
            pltpu.VMEM((rpt,), f32),
            pltpu.SemaphoreType.DMA,
        ],
    )
    def deg_kernel(src_hbm, dst_hbm, out_hbm,
                   stage, hsrc, hdst, sall, dall, gbuf, sumbuf, sem):
        c = lax.axis_index("c")
        s = lax.axis_index("s")
        wid = s * NC + c
        ebase = pl.multiple_of(wid * e_per_w, 8)

        a = pltpu.async_copy(src_hbm.at[pl.ds(ebase, e_per_w)], sall, sem)
        b = pltpu.async_copy(dst_hbm.at[pl.ds(ebase, e_per_w)], dall, sem)

        @pl.loop(0, n_pad // 16)
        def _(i):
            z = jnp.zeros((16,), f32)
            hsrc[pl.ds(i * 16, 16)] = z
            hdst[pl.ds(i * 16, 16)] = z

        a.wait()
        b.wait()

        def hist_step(buf, hist, off):
            idx = buf[pl.ds(off, 16)]
            cnt, last = plsc.scan_count(idx)
            plsc.addupdate_scatter(hist, [idx], cnt.astype(f32), mask=last)

        n4 = e_per_w // 64
        rem4 = e_per_w - n4 * 64

        @pl.loop(0, n4)
        def _(i):
            o = i * 64
            for u in range(0, 64, 16):
                hist_step(sall, hsrc, o + u)
                hist_step(dall, hdst, o + u)

        for r in range(0, rem4, 16):
            hist_step(sall, hsrc, n4 * 64 + r)
            hist_step(dall, hdst, n4 * 64 + r)

        pltpu.sync_copy(hsrc, stage.at[0, s])
        pltpu.sync_copy(hdst, stage.at[1, s])
        plsc.subcore_barrier()

        for which in range(2):
            cps = []
            for k in range(NS):
                cps.append(pltpu.async_copy(
                    stage.at[which, k, pl.ds(s * rpt, rpt)], gbuf.at[k], sem))
            for cp in cps:
                cp.wait()
            for v in range(rpt // 16):
                acc = gbuf[0, pl.ds(v * 16, 16)]
                for k in range(1, NS):
                    acc = acc + gbuf[k, pl.ds(v * 16, 16)]
                sumbuf[pl.ds(v * 16, 16)] = acc
            obase = pl.multiple_of((c * 2 + which) * n_pad + s * rpt, 8)
            pltpu.sync_copy(sumbuf, out_hbm.at[pl.ds(obase, rpt)])

    return deg_kernel


@functools.lru_cache(maxsize=None)
def _make_sc_aggregate(n_pad, width, n_edges, block, n_stage=0):
    e_per_w = n_edges // NW
    n_full = e_per_w // block
    tail = e_per_w - n_full * block
    assert block % 8 == 0 and block <= 128 and n_full % 2 == 0
    assert tail % 8 == 0 and (tail == 0 or tail >= 8)
    rpt = n_pad // NS
    nz = rpt // block
    assert nz * block == rpt
    mesh = plsc.VectorSubcoreMesh(
        core_axis_name="c", subcore_axis_name="s", num_cores=NC, num_subcores=NS
    )

    @functools.partial(
        pl.kernel,
        out_type=jax.ShapeDtypeStruct((NC, n_pad, width), f32),
        mesh=mesh,
        compiler_params=pltpu.CompilerParams(use_tc_tiling_on_sc=(width % 128 == 0)),
        scratch_types=[
            pltpu.VMEM_SHARED((n_pad, width), f32),
            pltpu.VMEM_SHARED((n_pad if n_stage else 8, width), f32),
            pltpu.VMEM((block, width), f32),
            pltpu.VMEM((block, width), f32),
            pltpu.VMEM((max(tail, 8), width), f32),
            pltpu.VMEM((block,), i32),
            pltpu.VMEM((block,), i32),
            pltpu.VMEM((block,), i32),
            pltpu.VMEM((block,), i32),
            pltpu.VMEM((max(tail, 8),), i32),
            pltpu.VMEM((max(tail, 8),), i32),
            pltpu.SemaphoreType.DMA,
            pltpu.SemaphoreType.DMA,
        ],
    )
    def agg_kernel(xs_hbm, src_hbm, dst_hbm, zeros_hbm, out_hbm,
                   acc, stage, rows0, rows1, rows_t, sidx0, didx0, sidx1, didx1,
                   sidx_t, didx_t, sem_g, sem_i):
        c = lax.axis_index("c")
        s = lax.axis_index("s")
        wid = s * NC + c

        pltpu.sync_copy(zeros_hbm, rows0)
        for j in range(nz):
            pltpu.sync_copy(rows0, acc.at[pl.ds(s * rpt + j * block, block)])
        if n_stage:
            spt = n_stage // NS
            sc_ch = spt
            while sc_ch > block:
                sc_ch //= 5
            assert spt % sc_ch == 0 and sc_ch > 0
            for j in range(spt // sc_ch):
                off = s * spt + j * sc_ch
                pltpu.sync_copy(xs_hbm.at[pl.ds(off, sc_ch)],
                                rows1.at[pl.ds(0, sc_ch)])
                pltpu.sync_copy(rows1.at[pl.ds(0, sc_ch)],
                                stage.at[pl.ds(off, sc_ch)])
        plsc.subcore_barrier()

        gsrc = stage if n_stage else xs_hbm
        ebase = wid * e_per_w

        def idx_load(i, sbuf, dbuf):
            base = pl.multiple_of(ebase + i * block, 8)
            a = pltpu.async_copy(src_hbm.at[pl.ds(base, block)], sbuf, sem_i)
            b = pltpu.async_copy(dst_hbm.at[pl.ds(base, block)], dbuf, sem_i)
            return a, b

        a, b = idx_load(0, sidx0, didx0)
        a.wait()
        b.wait()
        pltpu.async_copy(gsrc.at[sidx0], rows0, sem_g)

        @pl.loop(0, n_full // 2 - 1)
        def _(p):
            i0 = 2 * p
            a, b = idx_load(i0 + 1, sidx1, didx1)
            pltpu.make_async_copy(gsrc.at[sidx0], rows0, sem_g).wait()
            a.wait()
            b.wait()
            pltpu.async_copy(gsrc.at[sidx1], rows1, sem_g)
            pltpu.sync_copy(rows0, acc.at[didx0], add=True)

            a2, b2 = idx_load(i0 + 2, sidx0, didx0)
            pltpu.make_async_copy(gsrc.at[sidx1], rows1, sem_g).wait()
            a2.wait()
            b2.wait()
            pltpu.async_copy(gsrc.at[sidx0], rows0, sem_g)
            pltpu.sync_copy(rows1, acc.at[didx1], add=True)

        a, b = idx_load(n_full - 1, sidx1, didx1)
        pltpu.make_async_copy(gsrc.at[sidx0], rows0, sem_g).wait()
        a.wait()
        b.wait()
        pltpu.async_copy(gsrc.at[sidx1], rows1, sem_g)
        pltpu.sync_copy(rows0, acc.at[didx0], add=True)

        if tail:
            base = pl.multiple_of(ebase + n_full * block, 8)
            ta = pltpu.async_copy(src_hbm.at[pl.ds(base, tail)], sidx_t, sem_i)
            tb = pltpu.async_copy(dst_hbm.at[pl.ds(base, tail)], didx_t, sem_i)
        pltpu.make_async_copy(gsrc.at[sidx1], rows1, sem_g).wait()
        if tail:
            ta.wait()
            tb.wait()
            gt = pltpu.async_copy(gsrc.at[sidx_t], rows_t, sem_g)
        pltpu.sync_copy(rows1, acc.at[didx1], add=True)
        if tail:
            gt.wait()
            pltpu.sync_copy(rows_t, acc.at[didx_t], add=True)

        plsc.subcore_barrier()
        for j in range(nz):
            pltpu.sync_copy(acc.at[pl.ds(s * rpt + j * block, block)], rows0)
            pltpu.sync_copy(rows0, out_hbm.at[c, pl.ds(s * rpt + j * block, block)])

    return agg_kernel


def _tc_prep(x, d_out0, d_out1, d_in0, d_in1):
    n, d = x.shape

    def body(x_ref, a_ref, b_ref, c_ref, e_ref, xs_ref, nin_ref, nout_ref):
        deg_out = a_ref[...] + b_ref[...]
        deg_in = c_ref[...] + e_ref[...]
        nout = lax.rsqrt(jnp.maximum(deg_out, 1.0))
        nin = lax.rsqrt(jnp.maximum(deg_in, 1.0))
        xs_ref[...] = x_ref[...] * nout
        nin_ref[...] = nin
        nout_ref[...] = nout

    return pl.pallas_call(
        body,
        out_shape=(
            jax.ShapeDtypeStruct((n, d), f32),
            jax.ShapeDtypeStruct((n, 1), f32),
            jax.ShapeDtypeStruct((n, 1), f32),
        ),
    )(x, d_out0, d_out1, d_in0, d_in1)


def _tc_mid(ap, nin, nout, w1, b1, wop):
    n = nin.shape[0]
    wpad = wop.shape[1]

    def body(ap_ref, nin_ref, nout_ref, w1_ref, b1_ref, wop_ref, y_ref):
        agg = (ap_ref[0, :n, :] + ap_ref[1, :n, :]) * nin_ref[...]
        h = jnp.dot(agg, w1_ref[...], preferred_element_type=f32) + b1_ref[...]
        h = jnp.maximum(h, 0.0)
        mean = jnp.mean(h, axis=0, keepdims=True)
        var = jnp.mean(h * h, axis=0, keepdims=True) - mean * mean
        hbn = (h - mean) * lax.rsqrt(var + 1e-5)
        y2 = jnp.dot(hbn, wop_ref[...], preferred_element_type=f32)
        y_ref[...] = y2 * nout_ref[...]

    return pl.pallas_call(
        body,
        out_shape=jax.ShapeDtypeStruct((n, wpad), f32),
    )(ap, nin, nout, w1, b1, wop)


def _tc_final(a2p, nin, bfull):
    n = nin.shape[0]
    wpad = bfull.shape[1]

    def body(ap_ref, nin_ref, b_ref, out_ref):
        logits = (ap_ref[0, :n, :] + ap_ref[1, :n, :]) * nin_ref[...] + b_ref[...]
        m = jnp.max(logits, axis=1, keepdims=True)
        e = jnp.exp(logits - m)
        out_ref[...] = e / jnp.sum(e, axis=1, keepdims=True)

    return pl.pallas_call(
        body,
        out_shape=jax.ShapeDtypeStruct((n, wpad), f32),
    )(a2p, nin, bfull)


def kernel(in_feat, edge_index, W1, b1, W_out, b_out):
    n, d = in_feat.shape
    h = W1.shape[1]
    out_dim = W_out.shape[1]
    e = edge_index.shape[1]
    wpad = 16
    block = 128
    n_pad = ((n + (block * NS) - 1) // (block * NS)) * (block * NS)

    src = edge_index[0].astype(i32)
    dst = edge_index[1].astype(i32)

    zeros_d = jnp.zeros((block, d), f32)
    zeros_w = jnp.zeros((block, wpad), f32)

    degp = _make_sc_degrees(n_pad, e)(src, dst)
    degp = degp.reshape(NC, 2, n_pad)
    d_out0 = degp[0, 0, :n].reshape(n, 1)
    d_out1 = degp[1, 0, :n].reshape(n, 1)
    d_in0 = degp[0, 1, :n].reshape(n, 1)
    d_in1 = degp[1, 1, :n].reshape(n, 1)

    xs, nin, nout = _tc_prep(in_feat, d_out0, d_out1, d_in0, d_in1)

    ap = _make_sc_aggregate(n_pad, d, e, block)(xs, src, dst, zeros_d)

    wop = jnp.concatenate([W_out, jnp.zeros((h, wpad - out_dim), f32)], axis=1)
    y2s = _tc_mid(ap, nin, nout, W1, b1.reshape(1, h), wop)

    a2p = _make_sc_aggregate(n_pad, wpad, e, block, n_stage=n)(y2s, src, dst, zeros_w)

    bfull = jnp.concatenate(
        [b_out, jnp.full((wpad - out_dim,), -1e30, f32)]
    ).reshape(1, wpad)
    out16 = _tc_final(a2p, nin, bfull)
    return out16[:, :out_dim]
